# baseline (speedup 1.0000x reference)
import functools
import math

import jax
import jax.numpy as jnp
from jax import lax
from jax.experimental import pallas as pl
from jax.experimental.pallas import tpu as pltpu
from jax.experimental.pallas import tpu_sc as plsc

B, C, D = 1024, 100000, 64
HARD_NEG_K = 5
TAU_MIN, TAU_MAX = math.log(0.01), math.log(1.0)
BLK = 4096
NBLK = (C + BLK - 1) // BLK
NEG_INF = float("-inf")


def _sc_gather_build():
    info = plsc.get_sparse_core_info()
    nw = info.num_cores * info.num_subcores
    b_per_w = B // nw
    mesh = plsc.VectorSubcoreMesh(core_axis_name="c", subcore_axis_name="s")

    @functools.partial(
        pl.kernel,
        mesh=mesh,
        out_type=jax.ShapeDtypeStruct((B,), jnp.float32),
        scratch_types=[
            pltpu.VMEM((b_per_w,), jnp.int32),
            pltpu.VMEM((b_per_w,), jnp.float32),
            pltpu.SemaphoreType.DMA,
        ],
    )
    def gather_kernel(table_hbm, idx_hbm, out_hbm, idx_v, vals_v, sem):
        wid = lax.axis_index("s") * info.num_cores + lax.axis_index("c")
        base = wid * b_per_w
        pltpu.sync_copy(idx_hbm.at[pl.ds(base, b_per_w)], idx_v)
        pltpu.async_copy(table_hbm.at[idx_v], vals_v, sem).wait()
        pltpu.sync_copy(vals_v, out_hbm.at[pl.ds(base, b_per_w)])

    return gather_kernel


def _tc_body(feats_ref, labels_ref, lt_ref, protos_ref, out_ref,
             m_ref, s_ref, top_ref, corr_ref, fn_ref):
    k = pl.program_id(0)

    @pl.when(k == 0)
    def _init():
        m_ref[...] = jnp.full((B, 1), NEG_INF, dtype=jnp.float32)
        s_ref[...] = jnp.zeros((B, 1), dtype=jnp.float32)
        top_ref[...] = jnp.full((B, 8), NEG_INF, dtype=jnp.float32)
        corr_ref[...] = jnp.zeros((B, 1), dtype=jnp.float32)
        f = feats_ref[...]
        fn = f / jnp.maximum(
            jnp.sqrt(jnp.sum(f * f, axis=1, keepdims=True)), 1e-12)
        tau = jnp.exp(jnp.clip(lt_ref[...], TAU_MIN, TAU_MAX))
        fn_ref[...] = fn * (1.0 / tau)

    fn = fn_ref[...]
    p = protos_ref[...]
    pn = p * (1.0 / jnp.maximum(
        jnp.sqrt(jnp.sum(p * p, axis=1, keepdims=True)), 1e-12))
    y32 = lax.dot_general(fn, pn, (((1,), (1,)), ((), ())),
                          preferred_element_type=jnp.float32)
    y = y32.astype(jnp.bfloat16)

    col = k * BLK + lax.broadcasted_iota(jnp.int32, (1, BLK), 1)
    notvalid = col >= C
    lab = labels_ref[...]
    is_lab = lab == col

    cy = jnp.sum(jnp.where(is_lab, y, jnp.bfloat16(0.0)),
                 axis=1, keepdims=True).astype(jnp.float32)
    blk_end = jnp.minimum((k + 1) * BLK, C)
    has_lab = (lab >= k * BLK) & (lab < blk_end)
    corr_ref[...] += cy

    cand = jnp.where(is_lab | notvalid, jnp.bfloat16(NEG_INF), y)

    mxb = jnp.max(cand, axis=1, keepdims=True)
    mx1 = mxb.astype(jnp.float32)
    es = jnp.sum(jnp.exp((cand - mxb).astype(jnp.float32)),
                 axis=1, keepdims=True)
    lab_term = jnp.where(has_lab, cy, NEG_INF)
    m_old = m_ref[...]
    m_new = jnp.maximum(jnp.maximum(m_old, mx1), lab_term)
    s_ref[...] = (s_ref[...] * jnp.exp(m_old - m_new)
                  + es * jnp.exp(mx1 - m_new)
                  + jnp.where(has_lab, jnp.exp(cy - m_new), 0.0))
    m_ref[...] = m_new

    bvals = [mx1]
    candb = cand
    for _ in range(HARD_NEG_K - 1):
        candb = jnp.where(candb >= mxb, jnp.bfloat16(NEG_INF), candb)
        mxb = jnp.max(candb, axis=1, keepdims=True)
        bvals.append(mxb.astype(jnp.float32))

    a = [top_ref[:, j:j + 1] for j in range(HARD_NEG_K)]
    pos_inf = jnp.full((B, 1), float("inf"), dtype=jnp.float32)
    a = [pos_inf] + a
    b = [pos_inf] + bvals
    ninf = jnp.full((B, 1), NEG_INF, dtype=jnp.float32)

    def pick(lst, i):
        return lst[i] if i < len(lst) else ninf

    new_top = []
    for j in range(HARD_NEG_K):
        terms = []
        for i in range(j + 2):
            terms.append(jnp.minimum(pick(a, i), pick(b, j + 1 - i)))
        cj = terms[0]
        for t in terms[1:]:
            cj = jnp.maximum(cj, t)
        new_top.append(cj)
    top_ref[...] = jnp.concatenate(new_top + [ninf, ninf, ninf], axis=1)

    @pl.when(k == NBLK - 1)
    def _fin():
        logz = m_ref[...] + jnp.log(s_ref[...])
        t = top_ref[...]
        hard = (t[:, 0:1] + t[:, 1:2] + t[:, 2:3] + t[:, 3:4] + t[:, 4:5]) / 5.0
        hmax = jnp.max(hard, axis=0, keepdims=True)
        e = jnp.exp(hard - hmax)
        w = jnp.minimum(e / jnp.sum(e, axis=0, keepdims=True) * B, 5.0)
        loss_per = logz - corr_ref[...]
        out_ref[...] = jnp.sum(loss_per * w, axis=0, keepdims=True) / B


def _tc_main(features, labels_col, lt_col, protos):
    return pl.pallas_call(
        _tc_body,
        grid=(NBLK,),
        in_specs=[
            pl.BlockSpec((B, D), lambda k: (0, 0)),
            pl.BlockSpec((B, 1), lambda k: (0, 0)),
            pl.BlockSpec((B, 1), lambda k: (0, 0)),
            pl.BlockSpec((BLK, D), lambda k: (k, 0)),
        ],
        out_specs=pl.BlockSpec((1, 1), lambda k: (0, 0)),
        out_shape=jax.ShapeDtypeStruct((1, 1), jnp.float32),
        scratch_shapes=[
            pltpu.VMEM((B, 1), jnp.float32),
            pltpu.VMEM((B, 1), jnp.float32),
            pltpu.VMEM((B, 8), jnp.float32),
            pltpu.VMEM((B, 1), jnp.float32),
            pltpu.VMEM((B, D), jnp.float32),
        ],
    )(features, labels_col, lt_col, protos)


def kernel(features, labels, shadow_prototypes, log_tau):
    labels_i32 = labels.astype(jnp.int32)
    lt_g = _sc_gather_build()(log_tau, labels_i32)
    out = _tc_main(features.astype(jnp.float32),
                   labels_i32.reshape(B, 1),
                   lt_g.reshape(B, 1),
                   shadow_prototypes.astype(jnp.float32))
    return out[0, 0]


# speedup vs baseline: 1.2133x; 1.2133x over previous
import functools
import math

import jax
import jax.numpy as jnp
from jax import lax
from jax.experimental import pallas as pl
from jax.experimental.pallas import tpu as pltpu
from jax.experimental.pallas import tpu_sc as plsc

B, C, D = 1024, 100000, 64
HARD_NEG_K = 5
TAU_MIN, TAU_MAX = math.log(0.01), math.log(1.0)
BLK = 4096
NBLK = (C + BLK - 1) // BLK
NEG_INF = float("-inf")
PAD_AUG = 60.0


def _sc_gather_build():
    info = plsc.get_sparse_core_info()
    nw = info.num_cores * info.num_subcores
    b_per_w = B // nw
    mesh = plsc.VectorSubcoreMesh(core_axis_name="c", subcore_axis_name="s")

    @functools.partial(
        pl.kernel,
        mesh=mesh,
        out_type=jax.ShapeDtypeStruct((B,), jnp.float32),
        scratch_types=[
            pltpu.VMEM((b_per_w,), jnp.int32),
            pltpu.VMEM((b_per_w,), jnp.float32),
            pltpu.SemaphoreType.DMA,
        ],
    )
    def gather_kernel(table_hbm, idx_hbm, out_hbm, idx_v, vals_v, sem):
        wid = lax.axis_index("s") * info.num_cores + lax.axis_index("c")
        base = wid * b_per_w
        pltpu.sync_copy(idx_hbm.at[pl.ds(base, b_per_w)], idx_v)
        pltpu.async_copy(table_hbm.at[idx_v], vals_v, sem).wait()
        pltpu.sync_copy(vals_v, out_hbm.at[pl.ds(base, b_per_w)])

    return gather_kernel


def _tc_body(feats_ref, labels_ref, lt_ref, protos_ref, out_ref,
             s_ref, top_ref, corr_ref, fn_ref):
    k = pl.program_id(0)

    @pl.when(k == 0)
    def _init():
        s_ref[...] = jnp.zeros((B, 1), dtype=jnp.float32)
        top_ref[...] = jnp.full((B, 8), NEG_INF, dtype=jnp.float32)
        corr_ref[...] = jnp.zeros((B, 1), dtype=jnp.float32)
        f = feats_ref[...]
        fn = f / jnp.maximum(
            jnp.sqrt(jnp.sum(f * f, axis=1, keepdims=True)), 1e-12)
        tau = jnp.exp(jnp.clip(lt_ref[...], TAU_MIN, TAU_MAX))
        rtau = 1.0 / tau
        fn_ref[...] = jnp.concatenate([fn * rtau, -rtau], axis=1)

    fa = fn_ref[...]
    p = protos_ref[...]
    pn = p * (1.0 / jnp.maximum(
        jnp.sqrt(jnp.sum(p * p, axis=1, keepdims=True)), 1e-12))
    rowid = k * BLK + lax.broadcasted_iota(jnp.int32, (BLK, 1), 0)
    vrow = rowid < C
    pa = jnp.concatenate([jnp.where(vrow, pn, 0.0),
                          jnp.where(vrow, 1.0, PAD_AUG)],
                         axis=1)
    y = lax.dot_general(fa, pa, (((1,), (1,)), ((), ())),
                        preferred_element_type=jnp.float32)

    col = k * BLK + lax.broadcasted_iota(jnp.int32, (1, BLK), 1)
    lab = labels_ref[...]
    is_lab = lab == col

    cy = jnp.sum(jnp.where(is_lab, y, 0.0), axis=1, keepdims=True)
    blk_end = jnp.minimum((k + 1) * BLK, C)
    has_lab = (lab >= k * BLK) & (lab < blk_end)
    corr_ref[...] += cy

    cand = jnp.where(is_lab, NEG_INF, y)

    mx1 = jnp.max(cand, axis=1, keepdims=True)
    es = jnp.sum(jnp.exp(cand), axis=1, keepdims=True)
    s_ref[...] += es + jnp.where(has_lab, jnp.exp(cy), 0.0)

    bvals = [mx1]
    candb = cand.astype(jnp.bfloat16)
    mxb = mx1.astype(jnp.bfloat16)
    for _ in range(HARD_NEG_K - 1):
        candb = jnp.where(candb >= mxb, jnp.bfloat16(NEG_INF), candb)
        mxb = jnp.max(candb, axis=1, keepdims=True)
        bvals.append(mxb.astype(jnp.float32))

    a = [top_ref[:, j:j + 1] for j in range(HARD_NEG_K)]
    pos_inf = jnp.full((B, 1), float("inf"), dtype=jnp.float32)
    a = [pos_inf] + a
    b = [pos_inf] + bvals
    ninf = jnp.full((B, 1), NEG_INF, dtype=jnp.float32)

    def pick(lst, i):
        return lst[i] if i < len(lst) else ninf

    new_top = []
    for j in range(HARD_NEG_K):
        terms = []
        for i in range(j + 2):
            terms.append(jnp.minimum(pick(a, i), pick(b, j + 1 - i)))
        cj = terms[0]
        for t in terms[1:]:
            cj = jnp.maximum(cj, t)
        new_top.append(cj)
    top_ref[...] = jnp.concatenate(new_top + [ninf, ninf, ninf], axis=1)

    @pl.when(k == NBLK - 1)
    def _fin():
        shift = -fn_ref[:, D:D + 1]
        logz = jnp.log(s_ref[...]) + shift
        corr = corr_ref[...] + shift
        t = top_ref[...]
        hard = (t[:, 0:1] + t[:, 1:2] + t[:, 2:3] + t[:, 3:4]
                + t[:, 4:5]) / 5.0 + shift
        hmax = jnp.max(hard, axis=0, keepdims=True)
        e = jnp.exp(hard - hmax)
        w = jnp.minimum(e / jnp.sum(e, axis=0, keepdims=True) * B, 5.0)
        loss_per = logz - corr
        out_ref[...] = jnp.sum(loss_per * w, axis=0, keepdims=True) / B


def _tc_main(features, labels_col, lt_col, protos):
    return pl.pallas_call(
        _tc_body,
        grid=(NBLK,),
        in_specs=[
            pl.BlockSpec((B, D), lambda k: (0, 0)),
            pl.BlockSpec((B, 1), lambda k: (0, 0)),
            pl.BlockSpec((B, 1), lambda k: (0, 0)),
            pl.BlockSpec((BLK, D), lambda k: (k, 0)),
        ],
        out_specs=pl.BlockSpec((1, 1), lambda k: (0, 0)),
        out_shape=jax.ShapeDtypeStruct((1, 1), jnp.float32),
        scratch_shapes=[
            pltpu.VMEM((B, 1), jnp.float32),
            pltpu.VMEM((B, 8), jnp.float32),
            pltpu.VMEM((B, 1), jnp.float32),
            pltpu.VMEM((B, D + 1), jnp.float32),
        ],
    )(features, labels_col, lt_col, protos)


def kernel(features, labels, shadow_prototypes, log_tau):
    labels_i32 = labels.astype(jnp.int32)
    lt_g = _sc_gather_build()(log_tau, labels_i32)
    out = _tc_main(features.astype(jnp.float32),
                   labels_i32.reshape(B, 1),
                   lt_g.reshape(B, 1),
                   shadow_prototypes.astype(jnp.float32))
    return out[0, 0]


# speedup vs baseline: 1.2729x; 1.0492x over previous
import functools
import math

import jax
import jax.numpy as jnp
from jax import lax
from jax.experimental import pallas as pl
from jax.experimental.pallas import tpu as pltpu
from jax.experimental.pallas import tpu_sc as plsc

B, C, D = 1024, 100000, 64
HARD_NEG_K = 5
TAU_MIN, TAU_MAX = math.log(0.01), math.log(1.0)
BLK = 4096
NBLK = (C + BLK - 1) // BLK
NEG_INF = float("-inf")
PAD_AUG = 60.0


def _sc_gather_build():
    info = plsc.get_sparse_core_info()
    nw = info.num_cores * info.num_subcores
    b_per_w = B // nw
    mesh = plsc.VectorSubcoreMesh(core_axis_name="c", subcore_axis_name="s")

    @functools.partial(
        pl.kernel,
        mesh=mesh,
        out_type=jax.ShapeDtypeStruct((B,), jnp.float32),
        scratch_types=[
            pltpu.VMEM((b_per_w,), jnp.int32),
            pltpu.VMEM((b_per_w,), jnp.float32),
            pltpu.SemaphoreType.DMA,
        ],
    )
    def gather_kernel(table_hbm, idx_hbm, out_hbm, idx_v, vals_v, sem):
        wid = lax.axis_index("s") * info.num_cores + lax.axis_index("c")
        base = wid * b_per_w
        pltpu.sync_copy(idx_hbm.at[pl.ds(base, b_per_w)], idx_v)
        pltpu.async_copy(table_hbm.at[idx_v], vals_v, sem).wait()
        pltpu.sync_copy(vals_v, out_hbm.at[pl.ds(base, b_per_w)])

    return gather_kernel


def _tc_body(feats_ref, labels_ref, lt_ref, protos_ref, out_ref,
             s_ref, top_ref, corr_ref, fn_ref):
    k = pl.program_id(0)

    @pl.when(k == 0)
    def _init():
        s_ref[...] = jnp.zeros((B, 1), dtype=jnp.float32)
        top_ref[...] = jnp.full((B, 8), NEG_INF, dtype=jnp.float32)
        corr_ref[...] = jnp.zeros((B, 1), dtype=jnp.float32)
        f = feats_ref[...]
        fn = f / jnp.maximum(
            jnp.sqrt(jnp.sum(f * f, axis=1, keepdims=True)), 1e-12)
        tau = jnp.exp(jnp.clip(lt_ref[...], TAU_MIN, TAU_MAX))
        rtau = 1.0 / tau
        fn_ref[...] = jnp.concatenate([fn * rtau, -rtau], axis=1)

    fa = fn_ref[...]
    p = protos_ref[...]
    pn = p * (1.0 / jnp.maximum(
        jnp.sqrt(jnp.sum(p * p, axis=1, keepdims=True)), 1e-12))
    rowid = k * BLK + lax.broadcasted_iota(jnp.int32, (BLK, 1), 0)
    vrow = rowid < C
    pa = jnp.concatenate([jnp.where(vrow, pn, 0.0),
                          jnp.where(vrow, 1.0, PAD_AUG)],
                         axis=1)
    y = lax.dot_general(fa, pa, (((1,), (1,)), ((), ())),
                        preferred_element_type=jnp.float32)

    col = k * BLK + lax.broadcasted_iota(jnp.int32, (1, BLK), 1)
    lab = labels_ref[...]
    is_lab = lab == col

    cy = jnp.sum(jnp.where(is_lab, y, 0.0), axis=1, keepdims=True)
    corr_ref[...] += cy

    s_ref[...] += jnp.sum(jnp.exp(y), axis=1, keepdims=True)

    candb = jnp.where(is_lab, jnp.bfloat16(NEG_INF), y.astype(jnp.bfloat16))
    mxb = jnp.max(candb, axis=1, keepdims=True)
    bvals = [mxb.astype(jnp.float32)]
    for _ in range(HARD_NEG_K - 1):
        candb = jnp.where(candb >= mxb, jnp.bfloat16(NEG_INF), candb)
        mxb = jnp.max(candb, axis=1, keepdims=True)
        bvals.append(mxb.astype(jnp.float32))

    a = [top_ref[:, j:j + 1] for j in range(HARD_NEG_K)]
    pos_inf = jnp.full((B, 1), float("inf"), dtype=jnp.float32)
    a = [pos_inf] + a
    b = [pos_inf] + bvals
    ninf = jnp.full((B, 1), NEG_INF, dtype=jnp.float32)

    def pick(lst, i):
        return lst[i] if i < len(lst) else ninf

    new_top = []
    for j in range(HARD_NEG_K):
        terms = []
        for i in range(j + 2):
            terms.append(jnp.minimum(pick(a, i), pick(b, j + 1 - i)))
        cj = terms[0]
        for t in terms[1:]:
            cj = jnp.maximum(cj, t)
        new_top.append(cj)
    top_ref[...] = jnp.concatenate(new_top + [ninf, ninf, ninf], axis=1)

    @pl.when(k == NBLK - 1)
    def _fin():
        shift = -fn_ref[:, D:D + 1]
        logz = jnp.log(s_ref[...]) + shift
        corr = corr_ref[...] + shift
        t = top_ref[...]
        hard = (t[:, 0:1] + t[:, 1:2] + t[:, 2:3] + t[:, 3:4]
                + t[:, 4:5]) / 5.0 + shift
        hmax = jnp.max(hard, axis=0, keepdims=True)
        e = jnp.exp(hard - hmax)
        w = jnp.minimum(e / jnp.sum(e, axis=0, keepdims=True) * B, 5.0)
        loss_per = logz - corr
        out_ref[...] = jnp.sum(loss_per * w, axis=0, keepdims=True) / B


def _tc_main(features, labels_col, lt_col, protos):
    return pl.pallas_call(
        _tc_body,
        grid=(NBLK,),
        in_specs=[
            pl.BlockSpec((B, D), lambda k: (0, 0)),
            pl.BlockSpec((B, 1), lambda k: (0, 0)),
            pl.BlockSpec((B, 1), lambda k: (0, 0)),
            pl.BlockSpec((BLK, D), lambda k: (k, 0)),
        ],
        out_specs=pl.BlockSpec((1, 1), lambda k: (0, 0)),
        out_shape=jax.ShapeDtypeStruct((1, 1), jnp.float32),
        scratch_shapes=[
            pltpu.VMEM((B, 1), jnp.float32),
            pltpu.VMEM((B, 8), jnp.float32),
            pltpu.VMEM((B, 1), jnp.float32),
            pltpu.VMEM((B, D + 1), jnp.float32),
        ],
    )(features, labels_col, lt_col, protos)


def kernel(features, labels, shadow_prototypes, log_tau):
    labels_i32 = labels.astype(jnp.int32)
    lt_g = _sc_gather_build()(log_tau, labels_i32)
    out = _tc_main(features.astype(jnp.float32),
                   labels_i32.reshape(B, 1),
                   lt_g.reshape(B, 1),
                   shadow_prototypes.astype(jnp.float32))
    return out[0, 0]
